# baseline (speedup 1.0000x reference)
import functools

import jax
import jax.numpy as jnp
from jax import lax
from jax.experimental import pallas as pl
from jax.experimental.pallas import tpu as pltpu
from jax.experimental.pallas import tpu_sc as plsc

_IDX_CHUNK = 128


@functools.lru_cache(maxsize=None)
def _make_gather(V, D, B):
    info = plsc.get_sparse_core_info()
    NC, NS = info.num_cores, info.num_subcores
    NW = NC * NS
    assert B % (NW * _IDX_CHUNK) == 0 and D % info.num_lanes == 0
    b_per_w = B // NW
    n_chunks = b_per_w // _IDX_CHUNK

    mesh = plsc.VectorSubcoreMesh(core_axis_name="c", subcore_axis_name="s")

    @functools.partial(
        pl.kernel,
        mesh=mesh,
        out_type=jax.ShapeDtypeStruct((B, D), jnp.float32),
        scratch_types=[
            pltpu.VMEM((b_per_w,), jnp.int32),
            pltpu.VMEM((b_per_w, D), jnp.float32),
            pltpu.SemaphoreType.DMA,
        ],
        compiler_params=pltpu.CompilerParams(use_tc_tiling_on_sc=False),
    )
    def gather_kernel(table_hbm, idx_hbm, out_hbm, idx_v, rows_v, sem):
        wid = lax.axis_index("s") * NC + lax.axis_index("c")
        base = wid * b_per_w
        pltpu.sync_copy(idx_hbm.at[pl.ds(base, b_per_w)], idx_v)
        copies = []
        for j in range(n_chunks):
            copies.append(
                pltpu.async_copy(
                    table_hbm.at[idx_v.at[pl.ds(j * _IDX_CHUNK, _IDX_CHUNK)]],
                    rows_v.at[pl.ds(j * _IDX_CHUNK, _IDX_CHUNK)],
                    sem,
                )
            )
        for c in copies:
            c.wait()
        pltpu.sync_copy(rows_v, out_hbm.at[pl.ds(base, b_per_w)])

    return gather_kernel


def kernel(u_input, embedding_weight):
    V, D = embedding_weight.shape
    B = u_input.shape[0]
    idx = u_input.reshape(B)
    out = _make_gather(V, D, B)(embedding_weight, idx)
    return (out, out)


# speedup vs baseline: 3.4715x; 3.4715x over previous
import functools

import jax
import jax.numpy as jnp
from jax import lax
from jax.experimental import pallas as pl
from jax.experimental.pallas import tpu as pltpu
from jax.experimental.pallas import tpu_sc as plsc

_LANES = 16
_TCOL = 128


@functools.lru_cache(maxsize=None)
def _make_gather(V, D, B):
    info = plsc.get_sparse_core_info()
    NC, NS = info.num_cores, info.num_subcores
    NW = NC * NS
    assert B % (NW * _LANES) == 0 and D % _LANES == 0
    b_per_w = B // NW
    n_groups = b_per_w // _LANES
    n_j = D // _LANES

    mesh = plsc.VectorSubcoreMesh(core_axis_name="c", subcore_axis_name="s")

    @functools.partial(
        pl.kernel,
        mesh=mesh,
        out_type=jax.ShapeDtypeStruct((D, B), jnp.float32),
        scratch_types=[
            pltpu.VMEM((b_per_w + _LANES,), jnp.int32),
            pltpu.VMEM((_LANES, D, _TCOL), jnp.float32),
            pltpu.VMEM((D, b_per_w), jnp.float32),
            [pltpu.SemaphoreType.DMA] * _LANES,
            pltpu.SemaphoreType.DMA,
        ],
        compiler_params=pltpu.CompilerParams(needs_layout_passes=False),
    )
    def gather_kernel(tT_hbm, idx_hbm, out_hbm, idx_v, ring_v, out_v, sems, osem):
        wid = lax.axis_index("s") * NC + lax.axis_index("c")
        base = wid * b_per_w
        pltpu.sync_copy(
            idx_hbm.at[pl.ds(base, b_per_w)], idx_v.at[pl.ds(0, b_per_w)]
        )
        jrows = [
            lax.iota(jnp.int32, _LANES) + (j * _LANES) for j in range(n_j)
        ]

        def fetch_group(g):
            qoff = idx_v[pl.ds(g * _LANES, _LANES)] & jnp.int32(~(_TCOL - 1))
            for k in range(_LANES):
                o = pl.multiple_of(qoff[k], _TCOL)
                pltpu.async_copy(
                    tT_hbm.at[:, pl.ds(o, _TCOL)], ring_v.at[k], sems[k]
                )

        def extract_group(g):
            vec = idx_v[pl.ds(g * _LANES, _LANES)]
            rvec = vec & jnp.int32(_TCOL - 1)
            for k in range(_LANES):
                pltpu.make_async_copy(
                    tT_hbm.at[:, pl.ds(0, _TCOL)], ring_v.at[k], sems[k]
                ).wait()
                col = jnp.full((_LANES,), rvec[k], dtype=jnp.int32)
                bcol = jnp.full((_LANES,), g * _LANES + k, dtype=jnp.int32)
                for j in range(n_j):
                    vals = plsc.load_gather(ring_v.at[k], [jrows[j], col])
                    plsc.store_scatter(out_v, [jrows[j], bcol], vals)

        fetch_group(0)

        def body(g, carry):
            extract_group(g)

            @pl.when(g + 1 < n_groups)
            def _():
                fetch_group(g + 1)

            return carry

        lax.fori_loop(0, n_groups, body, 0)
        pltpu.async_copy(out_v, out_hbm.at[:, pl.ds(base, b_per_w)], osem).wait()

    return gather_kernel


def kernel(u_input, embedding_weight):
    V, D = embedding_weight.shape
    B = u_input.shape[0]
    idx = u_input.reshape(B)
    outT = _make_gather(V, D, B)(embedding_weight.T, idx)
    out = outT.T
    return (out, out)


# speedup vs baseline: 3.9308x; 1.1323x over previous
import functools

import jax
import jax.numpy as jnp
from jax import lax
from jax.experimental import pallas as pl
from jax.experimental.pallas import tpu as pltpu
from jax.experimental.pallas import tpu_sc as plsc

_LANES = 16
_TCOL = 128


@functools.lru_cache(maxsize=None)
def _make_gather(V, D, B):
    info = plsc.get_sparse_core_info()
    NC, NS = info.num_cores, info.num_subcores
    NW = NC * NS
    assert B % (NW * _LANES) == 0 and D % _LANES == 0
    b_per_w = B // NW
    n_groups = b_per_w // _LANES
    n_j = D // _LANES

    mesh = plsc.VectorSubcoreMesh(core_axis_name="c", subcore_axis_name="s")

    @functools.partial(
        pl.kernel,
        mesh=mesh,
        out_type=jax.ShapeDtypeStruct((D, B), jnp.float32),
        scratch_types=[
            pltpu.VMEM((b_per_w + _LANES,), jnp.int32),
            pltpu.VMEM((_LANES, D, _TCOL), jnp.float32),
            pltpu.VMEM((D, b_per_w), jnp.float32),
            [pltpu.SemaphoreType.DMA] * _LANES,
            pltpu.SemaphoreType.DMA,
        ],
        compiler_params=pltpu.CompilerParams(needs_layout_passes=False),
    )
    def gather_kernel(tT_hbm, idx_hbm, out_hbm, idx_v, ring_v, out_v, sems, osem):
        wid = lax.axis_index("s") * NC + lax.axis_index("c")
        base = wid * b_per_w
        pltpu.sync_copy(
            idx_hbm.at[pl.ds(base, b_per_w)], idx_v.at[pl.ds(0, b_per_w)]
        )
        jrows = [
            lax.iota(jnp.int32, _LANES) + (j * _LANES) for j in range(n_j)
        ]

        def fetch_group(g):
            qoff = idx_v[pl.ds(g * _LANES, _LANES)] & jnp.int32(~(_TCOL - 1))
            for k in range(_LANES):
                o = pl.multiple_of(qoff[k], _TCOL)
                pltpu.async_copy(
                    tT_hbm.at[:, pl.ds(o, _TCOL)], ring_v.at[k], sems[k]
                )

        fetch_group(0)

        def body(g, carry):
            vec = idx_v[pl.ds(g * _LANES, _LANES)]
            rvec = vec & jnp.int32(_TCOL - 1)
            qoff_nxt = (
                idx_v[pl.ds((g + 1) * _LANES, _LANES)] & jnp.int32(~(_TCOL - 1))
            )
            for k in range(_LANES):
                pltpu.make_async_copy(
                    tT_hbm.at[:, pl.ds(0, _TCOL)], ring_v.at[k], sems[k]
                ).wait()
                col = jnp.full((_LANES,), rvec[k], dtype=jnp.int32)
                bcol = jnp.full((_LANES,), g * _LANES + k, dtype=jnp.int32)
                for j in range(n_j):
                    vals = plsc.load_gather(ring_v.at[k], [jrows[j], col])
                    plsc.store_scatter(out_v, [jrows[j], bcol], vals)

                @pl.when(g + 1 < n_groups)
                def _():
                    o = pl.multiple_of(qoff_nxt[k], _TCOL)
                    pltpu.async_copy(
                        tT_hbm.at[:, pl.ds(o, _TCOL)], ring_v.at[k], sems[k]
                    )

            return carry

        lax.fori_loop(0, n_groups, body, 0)
        pltpu.async_copy(out_v, out_hbm.at[:, pl.ds(base, b_per_w)], osem).wait()

    return gather_kernel


def kernel(u_input, embedding_weight):
    V, D = embedding_weight.shape
    B = u_input.shape[0]
    idx = u_input.reshape(B)
    outT = _make_gather(V, D, B)(embedding_weight.T, idx)
    out = outT.T
    return (out, out)


# speedup vs baseline: 4.0238x; 1.0237x over previous
import functools

import jax
import jax.numpy as jnp
from jax import lax
from jax.experimental import pallas as pl
from jax.experimental.pallas import tpu as pltpu
from jax.experimental.pallas import tpu_sc as plsc

_LANES = 16
_TCOL = 128


@functools.lru_cache(maxsize=None)
def _make_gather(V, D, B):
    info = plsc.get_sparse_core_info()
    NC, NS = info.num_cores, info.num_subcores
    NW = NC * NS
    assert B % (NW * _LANES) == 0 and D % _LANES == 0
    b_per_w = B // NW
    n_groups = b_per_w // _LANES
    n_j = D // _LANES

    mesh = plsc.VectorSubcoreMesh(core_axis_name="c", subcore_axis_name="s")

    @functools.partial(
        pl.kernel,
        mesh=mesh,
        out_type=(
            jax.ShapeDtypeStruct((D, B), jnp.float32),
            jax.ShapeDtypeStruct((D, B), jnp.float32),
        ),
        scratch_types=[
            pltpu.VMEM((b_per_w + _LANES,), jnp.int32),
            pltpu.VMEM((_LANES, D, _TCOL), jnp.float32),
            pltpu.VMEM((D, b_per_w), jnp.float32),
            [pltpu.SemaphoreType.DMA] * _LANES,
            pltpu.SemaphoreType.DMA,
        ],
        compiler_params=pltpu.CompilerParams(needs_layout_passes=False),
    )
    def gather_kernel(tT_hbm, idx_hbm, o1_hbm, o2_hbm, idx_v, ring_v, out_v, sems, osem):
        wid = lax.axis_index("s") * NC + lax.axis_index("c")
        base = wid * b_per_w
        pltpu.sync_copy(
            idx_hbm.at[pl.ds(base, b_per_w)], idx_v.at[pl.ds(0, b_per_w)]
        )
        jrows = [
            lax.iota(jnp.int32, _LANES) + (j * _LANES) for j in range(n_j)
        ]

        def fetch_group(g):
            qoff = idx_v[pl.ds(g * _LANES, _LANES)] & jnp.int32(~(_TCOL - 1))
            for k in range(_LANES):
                o = pl.multiple_of(qoff[k], _TCOL)
                pltpu.async_copy(
                    tT_hbm.at[:, pl.ds(o, _TCOL)], ring_v.at[k], sems[k]
                )

        fetch_group(0)

        def body(g, carry):
            vec = idx_v[pl.ds(g * _LANES, _LANES)]
            rvec = vec & jnp.int32(_TCOL - 1)
            qoff_nxt = (
                idx_v[pl.ds((g + 1) * _LANES, _LANES)] & jnp.int32(~(_TCOL - 1))
            )
            for k in range(_LANES):
                pltpu.make_async_copy(
                    tT_hbm.at[:, pl.ds(0, _TCOL)], ring_v.at[k], sems[k]
                ).wait()
                col = jnp.full((_LANES,), rvec[k], dtype=jnp.int32)
                bcol = jnp.full((_LANES,), g * _LANES + k, dtype=jnp.int32)
                for j in range(n_j):
                    vals = plsc.load_gather(ring_v.at[k], [jrows[j], col])
                    plsc.store_scatter(out_v, [jrows[j], bcol], vals)

                @pl.when(g + 1 < n_groups)
                def _():
                    o = pl.multiple_of(qoff_nxt[k], _TCOL)
                    pltpu.async_copy(
                        tT_hbm.at[:, pl.ds(o, _TCOL)], ring_v.at[k], sems[k]
                    )

            return carry

        lax.fori_loop(0, n_groups, body, 0)
        c1 = pltpu.async_copy(out_v, o1_hbm.at[:, pl.ds(base, b_per_w)], osem)
        c2 = pltpu.async_copy(out_v, o2_hbm.at[:, pl.ds(base, b_per_w)], osem)
        c1.wait()
        c2.wait()

    return gather_kernel


def kernel(u_input, embedding_weight):
    V, D = embedding_weight.shape
    B = u_input.shape[0]
    idx = u_input.reshape(B)
    o1, o2 = _make_gather(V, D, B)(embedding_weight.T, idx)
    return (o1.T, o2.T)
